# Initial kernel scaffold; baseline (speedup 1.0000x reference)
import jax
import jax.numpy as jnp
from jax.experimental import pallas as pl


def kernel(x_features, x_attention):
    raise NotImplementedError("write your pallas kernel here")


# speedup vs baseline: 7.8020x; 7.8020x over previous
import functools

import jax
import jax.numpy as jnp
from jax import lax
from jax.experimental import pallas as pl
from jax.experimental.pallas import tpu as pltpu
from jax.experimental.pallas import tpu_sc as plsc

N = 100000
D = 128
TOPK = 16
L = 16
NC = 2
NS = 16
NW = NC * NS
PW = 3136
NPAD = NW * PW
CPW = PW // L
BIG = float(jnp.finfo(jnp.float32).max)
IMAX = 2**31 - 1

_mesh = plsc.VectorSubcoreMesh(core_axis_name="c", subcore_axis_name="s")


def _merge16(av, ai, xv, xi, descending):
  rv = lax.rev(xv, (0,))
  ri = lax.rev(xi, (0,))
  if descending:
    take = (av > rv) | ((av == rv) & (ai < ri))
  else:
    take = (av < rv) | ((av == rv) & (ai < ri))
  mv = jnp.where(take, av, rv)
  mi = jnp.where(take, ai, ri)
  sv, si = plsc.sort_key_val(mv, mi, descending=descending)
  return sv, si


@functools.partial(
    pl.kernel,
    out_type=(
        jax.ShapeDtypeStruct((NW * L,), jnp.float32),
        jax.ShapeDtypeStruct((NW * L,), jnp.int32),
        jax.ShapeDtypeStruct((NW * L,), jnp.float32),
        jax.ShapeDtypeStruct((NW * L,), jnp.int32),
    ),
    mesh=_mesh,
    compiler_params=pltpu.CompilerParams(needs_layout_passes=False),
    scratch_types=[
        pltpu.VMEM((PW,), jnp.float32),
        pltpu.VMEM((L,), jnp.float32),
        pltpu.VMEM((L,), jnp.int32),
        pltpu.VMEM((L,), jnp.float32),
        pltpu.VMEM((L,), jnp.int32),
    ],
)
def _scan_candidates(att_hbm, topv_hbm, topi_hbm, botv_hbm, boti_hbm,
                     att_v, tv_s, ti_s, bv_s, bi_s):
  cid = lax.axis_index("c")
  sid = lax.axis_index("s")
  wid = sid * NC + cid
  base = wid * PW
  pltpu.sync_copy(att_hbm.at[pl.ds(base, PW)], att_v)

  iota = lax.iota(jnp.int32, L)
  tv0 = jnp.full((L,), -BIG, jnp.float32)
  bv0 = jnp.full((L,), BIG, jnp.float32)
  i0 = jnp.full((L,), IMAX, jnp.int32)

  def body(c, carry):
    tv, ti, bv, bi = carry
    v = att_v[pl.ds(c * L, L)]
    idx = iota + (base + c * L)
    sv, si = plsc.sort_key_val(v, idx, descending=True)
    tv, ti = _merge16(tv, ti, sv, si, descending=True)
    vb = jnp.where(idx < N, v, BIG)
    sv2, si2 = plsc.sort_key_val(vb, idx, descending=False)
    bv, bi = _merge16(bv, bi, sv2, si2, descending=False)
    return tv, ti, bv, bi

  tv, ti, bv, bi = lax.fori_loop(0, CPW, body, (tv0, i0, bv0, i0))

  tv_s[...] = tv
  ti_s[...] = ti
  bv_s[...] = bv
  bi_s[...] = bi
  out = pl.ds(wid * L, L)
  pltpu.sync_copy(tv_s, topv_hbm.at[out])
  pltpu.sync_copy(ti_s, topi_hbm.at[out])
  pltpu.sync_copy(bv_s, botv_hbm.at[out])
  pltpu.sync_copy(bi_s, boti_hbm.at[out])


@functools.partial(
    pl.kernel,
    out_type=(
        jax.ShapeDtypeStruct((2 * TOPK,), jnp.float32),
        jax.ShapeDtypeStruct((2 * TOPK, D), jnp.float32),
    ),
    mesh=_mesh,
    compiler_params=pltpu.CompilerParams(needs_layout_passes=False),
    scratch_types=[
        pltpu.VMEM((NW * L,), jnp.float32),
        pltpu.VMEM((NW * L,), jnp.int32),
        pltpu.VMEM((NW * L,), jnp.float32),
        pltpu.VMEM((NW * L,), jnp.int32),
        pltpu.VMEM((2 * TOPK,), jnp.int32),
        pltpu.VMEM((2 * TOPK,), jnp.float32),
        pltpu.VMEM((2 * TOPK, D), jnp.float32),
        pltpu.SemaphoreType.DMA,
    ],
)
def _merge_and_gather(feat_hbm, topv_hbm, topi_hbm, botv_hbm, boti_hbm,
                      att_out_hbm, feat_out_hbm,
                      tv_v, ti_v, bv_v, bi_v, idx_v, att_v, rows_v, sem):
  cid = lax.axis_index("c")
  sid = lax.axis_index("s")

  @pl.when(jnp.logical_and(cid == 0, sid == 0))
  def _():
    pltpu.sync_copy(topv_hbm, tv_v)
    pltpu.sync_copy(topi_hbm, ti_v)
    pltpu.sync_copy(botv_hbm, bv_v)
    pltpu.sync_copy(boti_hbm, bi_v)

    def merge_top(r, carry):
      av, ai = carry
      xv = tv_v[pl.ds(r * L, L)]
      xi = ti_v[pl.ds(r * L, L)]
      return _merge16(av, ai, xv, xi, descending=True)

    def merge_bot(r, carry):
      av, ai = carry
      xv = bv_v[pl.ds(r * L, L)]
      xi = bi_v[pl.ds(r * L, L)]
      return _merge16(av, ai, xv, xi, descending=False)

    tv, ti = lax.fori_loop(
        1, NW, merge_top, (tv_v[pl.ds(0, L)], ti_v[pl.ds(0, L)]))
    bv, bi = lax.fori_loop(
        1, NW, merge_bot, (bv_v[pl.ds(0, L)], bi_v[pl.ds(0, L)]))

    att_v[pl.ds(0, L)] = tv
    att_v[pl.ds(L, L)] = lax.rev(bv, (0,))
    idx_v[pl.ds(0, L)] = ti
    idx_v[pl.ds(L, L)] = lax.rev(bi, (0,))

    pltpu.async_copy(feat_hbm.at[idx_v], rows_v, sem).wait()
    pltpu.sync_copy(att_v, att_out_hbm)
    pltpu.sync_copy(rows_v, feat_out_hbm)


def kernel(x_features, x_attention):
  att = jnp.squeeze(x_attention, -1)
  att_pad = jnp.concatenate(
      [att, jnp.full((NPAD - N,), -BIG, jnp.float32)])
  topv, topi, botv, boti = _scan_candidates(att_pad)
  sel_att, sel_feat = _merge_and_gather(x_features, topv, topi, botv, boti)
  return sel_att.reshape(2 * TOPK, 1), sel_feat
